# Initial kernel scaffold; baseline (speedup 1.0000x reference)
import jax
import jax.numpy as jnp
from jax.experimental import pallas as pl


def kernel(feat_invar, feat_var, last_nodes, Wu, bu, Wv, We):
    raise NotImplementedError("write your pallas kernel here")


# speedup vs baseline: 9.3472x; 9.3472x over previous
import functools

import jax
import jax.numpy as jnp
from jax import lax
from jax.experimental import pallas as pl
from jax.experimental.pallas import tpu as pltpu
from jax.experimental.pallas import tpu_sc as plsc

B = 1000
NPG = 100
N = B * NPG
D = 128
H = 128

G = 8
BP = 1024


def _sc_gather(fi_hbm, fv_hbm, idx_hbm, oi_hbm, ov_hbm,
               idx_v, rows_i, rows_v, sem_i, sem_v):
    nc = plsc.get_sparse_core_info().num_cores
    wid = lax.axis_index("s") * nc + lax.axis_index("c")
    bpw = BP // (nc * plsc.get_sparse_core_info().num_subcores)
    base = wid * bpw
    pltpu.sync_copy(idx_hbm.at[pl.ds(base, bpw)], idx_v)
    ci = pltpu.async_copy(fi_hbm.at[idx_v], rows_i, sem_i)
    cv = pltpu.async_copy(fv_hbm.at[idx_v], rows_v, sem_v)
    ci.wait()
    cv.wait()
    pltpu.sync_copy(rows_i, oi_hbm.at[pl.ds(base, bpw)])
    pltpu.sync_copy(rows_v, ov_hbm.at[pl.ds(base, bpw)])


def _gather_last_rows(feat_invar, feat_var, idx_pad):
    info = plsc.get_sparse_core_info()
    bpw = BP // (info.num_cores * info.num_subcores)
    mesh = plsc.VectorSubcoreMesh(core_axis_name="c", subcore_axis_name="s")
    k = functools.partial(
        pl.kernel, mesh=mesh,
        out_type=[jax.ShapeDtypeStruct((BP, D), jnp.float32),
                  jax.ShapeDtypeStruct((BP, D), jnp.float32)],
        scratch_types=[
            pltpu.VMEM((bpw,), jnp.int32),
            pltpu.VMEM((bpw, D), jnp.float32),
            pltpu.VMEM((bpw, D), jnp.float32),
            pltpu.SemaphoreType.DMA,
            pltpu.SemaphoreType.DMA,
        ],
    )(_sc_gather)
    return k(feat_invar, feat_var, idx_pad)


def _attn_body(fi_ref, fv_ref, gvi_ref, gvv_ref, wu_ref, bu_ref, wv_ref,
               we_ref, oi_ref, ov_ref):
    Wu = wu_ref[...]
    bu = bu_ref[...]
    We = we_ref[...]
    Vi = jnp.dot(gvi_ref[...], wv_ref[...], preferred_element_type=jnp.float32)
    Vv = jnp.dot(gvv_ref[...], wv_ref[...], preferred_element_type=jnp.float32)
    for g in range(G):
        Xi = fi_ref[g]
        Xv = fv_ref[g]
        Ui = jnp.dot(Xi, Wu, preferred_element_type=jnp.float32) + bu
        Uv = jnp.dot(Xv, Wu, preferred_element_type=jnp.float32) + bu
        for v, oref in ((Vi[g:g + 1, :], oi_ref), (Vv[g:g + 1, :], ov_ref)):
            e1 = jnp.dot(jax.nn.sigmoid(Ui + v), We,
                         preferred_element_type=jnp.float32)
            e2 = jnp.dot(jax.nn.sigmoid(Uv + v), We,
                         preferred_element_type=jnp.float32)
            m = jnp.maximum(jnp.max(e1), jnp.max(e2))
            x1 = jnp.exp(e1 - m)
            x2 = jnp.exp(e2 - m)
            s = jnp.sum(x1) + jnp.sum(x2)
            r = (lax.dot_general(x1, Xi, (((0,), (0,)), ((), ())),
                                 preferred_element_type=jnp.float32) +
                 lax.dot_general(x2, Xv, (((0,), (0,)), ((), ())),
                                 preferred_element_type=jnp.float32))
            oref[g:g + 1, :] = r / s


def _attn_readout(fi3, fv3, gvi, gvv, Wu, bu2, Wv, We):
    return pl.pallas_call(
        _attn_body,
        grid=(B // G,),
        in_specs=[
            pl.BlockSpec((G, NPG, D), lambda i: (i, 0, 0)),
            pl.BlockSpec((G, NPG, D), lambda i: (i, 0, 0)),
            pl.BlockSpec((G, D), lambda i: (i, 0)),
            pl.BlockSpec((G, D), lambda i: (i, 0)),
            pl.BlockSpec((D, H), lambda i: (0, 0)),
            pl.BlockSpec((1, H), lambda i: (0, 0)),
            pl.BlockSpec((D, H), lambda i: (0, 0)),
            pl.BlockSpec((H, 1), lambda i: (0, 0)),
        ],
        out_specs=[pl.BlockSpec((G, D), lambda i: (i, 0)),
                   pl.BlockSpec((G, D), lambda i: (i, 0))],
        out_shape=[jax.ShapeDtypeStruct((B, D), jnp.float32),
                   jax.ShapeDtypeStruct((B, D), jnp.float32)],
    )(fi3, fv3, gvi, gvv, Wu, bu2, Wv, We)


def kernel(feat_invar, feat_var, last_nodes, Wu, bu, Wv, We):
    idx = last_nodes.astype(jnp.int32)
    idx_pad = jnp.concatenate([idx, jnp.zeros((BP - B,), jnp.int32)])
    gi, gv = _gather_last_rows(feat_invar, feat_var, idx_pad)
    fi3 = feat_invar.reshape(B, NPG, D)
    fv3 = feat_var.reshape(B, NPG, D)
    bu2 = bu.reshape(1, H)
    ri, rv = _attn_readout(fi3, fv3, gi[:B], gv[:B], Wu, bu2, Wv, We)
    return ri[:, None, :], rv[:, None, :]


# speedup vs baseline: 25.0935x; 2.6846x over previous
import functools

import jax
import jax.numpy as jnp
from jax import lax
from jax.experimental import pallas as pl
from jax.experimental.pallas import tpu as pltpu
from jax.experimental.pallas import tpu_sc as plsc

B = 1000
NPG = 100
N = B * NPG
D = 128
H = 128

G = 10
BP = 1024


def _sc_gather(fi_hbm, fv_hbm, idx_hbm, oi_hbm, ov_hbm,
               idx_v, rows_i, rows_v, sem_i, sem_v):
    nc = plsc.get_sparse_core_info().num_cores
    wid = lax.axis_index("s") * nc + lax.axis_index("c")
    bpw = BP // (nc * plsc.get_sparse_core_info().num_subcores)
    base = wid * bpw
    pltpu.sync_copy(idx_hbm.at[pl.ds(base, bpw)], idx_v)
    ci = pltpu.async_copy(fi_hbm.at[idx_v], rows_i, sem_i)
    cv = pltpu.async_copy(fv_hbm.at[idx_v], rows_v, sem_v)
    ci.wait()
    cv.wait()
    pltpu.sync_copy(rows_i, oi_hbm.at[pl.ds(base, bpw)])
    pltpu.sync_copy(rows_v, ov_hbm.at[pl.ds(base, bpw)])


def _gather_last_rows(feat_invar, feat_var, idx_pad):
    info = plsc.get_sparse_core_info()
    bpw = BP // (info.num_cores * info.num_subcores)
    mesh = plsc.VectorSubcoreMesh(core_axis_name="c", subcore_axis_name="s")
    k = functools.partial(
        pl.kernel, mesh=mesh,
        out_type=[jax.ShapeDtypeStruct((BP, D), jnp.float32),
                  jax.ShapeDtypeStruct((BP, D), jnp.float32)],
        scratch_types=[
            pltpu.VMEM((bpw,), jnp.int32),
            pltpu.VMEM((bpw, D), jnp.float32),
            pltpu.VMEM((bpw, D), jnp.float32),
            pltpu.SemaphoreType.DMA,
            pltpu.SemaphoreType.DMA,
        ],
    )(_sc_gather)
    return k(feat_invar, feat_var, idx_pad)


def _sigm(z):
    return 1.0 / (1.0 + jnp.exp(-z))


def _attn_body(fi_ref, fv_ref, gvi_ref, gvv_ref, wu_ref, bu_ref, wv_ref,
               we_ref, s_ref, oi_ref, ov_ref):
    f32 = jnp.float32
    bf = jnp.bfloat16
    dn = (((0,), (0,)), ((), ()))
    Xi = fi_ref[...]
    Xv = fv_ref[...]
    Xi_b = Xi.astype(bf)
    Xv_b = Xv.astype(bf)
    Wu_b = wu_ref[...].astype(bf)
    bu = bu_ref[...]
    Ui = jnp.dot(Xi_b, Wu_b, preferred_element_type=f32) + bu
    Uv = jnp.dot(Xv_b, Wu_b, preferred_element_type=f32) + bu
    Wv_b = wv_ref[...].astype(bf)
    Vi = jnp.dot(gvi_ref[0].astype(bf), Wv_b, preferred_element_type=f32)
    Vv = jnp.dot(gvv_ref[0].astype(bf), Wv_b, preferred_element_type=f32)
    S = s_ref[...]
    S_b = S.astype(bf)
    Vbi = jnp.dot(S_b, Vi.astype(bf), preferred_element_type=f32)
    Vbv = jnp.dot(S_b, Vv.astype(bf), preferred_element_type=f32)
    We = we_ref[...]
    m = jnp.sum(jnp.abs(We))
    We_b = We.astype(bf)
    e_ii = jnp.dot(_sigm(Ui + Vbi).astype(bf), We_b,
                   preferred_element_type=f32)
    e_vi = jnp.dot(_sigm(Uv + Vbi).astype(bf), We_b,
                   preferred_element_type=f32)
    e_iv = jnp.dot(_sigm(Ui + Vbv).astype(bf), We_b,
                   preferred_element_type=f32)
    e_vv = jnp.dot(_sigm(Uv + Vbv).astype(bf), We_b,
                   preferred_element_type=f32)
    x_ii = jnp.exp(e_ii - m)
    x_vi = jnp.exp(e_vi - m)
    x_iv = jnp.exp(e_iv - m)
    x_vv = jnp.exp(e_vv - m)
    A_i = jnp.concatenate([S * x_ii, S * x_iv], axis=1).astype(bf)
    A_v = jnp.concatenate([S * x_vi, S * x_vv], axis=1).astype(bf)
    Rp = (lax.dot_general(A_i, Xi_b, dn, preferred_element_type=f32) +
          lax.dot_general(A_v, Xv_b, dn, preferred_element_type=f32))
    xs = jnp.concatenate([x_ii + x_vi, x_iv + x_vv], axis=1)
    sp = lax.dot_general(S, xs, dn, preferred_element_type=f32)
    oi_ref[0] = Rp[:G] / sp[:, 0:1]
    ov_ref[0] = Rp[G:] / sp[:, 1:2]


def _attn_readout(fi, fv, gvi3, gvv3, Wu, bu2, Wv, We, S):
    R = G * NPG
    return pl.pallas_call(
        _attn_body,
        grid=(B // G,),
        in_specs=[
            pl.BlockSpec((R, D), lambda i: (i, 0)),
            pl.BlockSpec((R, D), lambda i: (i, 0)),
            pl.BlockSpec((1, G, D), lambda i: (i, 0, 0)),
            pl.BlockSpec((1, G, D), lambda i: (i, 0, 0)),
            pl.BlockSpec((D, H), lambda i: (0, 0)),
            pl.BlockSpec((1, H), lambda i: (0, 0)),
            pl.BlockSpec((D, H), lambda i: (0, 0)),
            pl.BlockSpec((H, 1), lambda i: (0, 0)),
            pl.BlockSpec((R, G), lambda i: (0, 0)),
        ],
        out_specs=[pl.BlockSpec((1, G, D), lambda i: (i, 0, 0)),
                   pl.BlockSpec((1, G, D), lambda i: (i, 0, 0))],
        out_shape=[jax.ShapeDtypeStruct((B // G, G, D), jnp.float32),
                   jax.ShapeDtypeStruct((B // G, G, D), jnp.float32)],
    )(fi, fv, gvi3, gvv3, Wu, bu2, Wv, We, S)


def kernel(feat_invar, feat_var, last_nodes, Wu, bu, Wv, We):
    idx = last_nodes.astype(jnp.int32)
    idx_pad = jnp.concatenate([idx, jnp.zeros((BP - B,), jnp.int32)])
    gi, gv = _gather_last_rows(feat_invar, feat_var, idx_pad)
    bu2 = bu.reshape(1, H)
    S = jnp.repeat(jnp.eye(G, dtype=jnp.float32), NPG, axis=0)
    ri, rv = _attn_readout(feat_invar, feat_var,
                           gi[:B].reshape(B // G, G, D),
                           gv[:B].reshape(B // G, G, D),
                           Wu, bu2, Wv, We, S)
    return (ri.reshape(B, D)[:, None, :], rv.reshape(B, D)[:, None, :])


# speedup vs baseline: 29.4197x; 1.1724x over previous
import functools

import jax
import jax.numpy as jnp
from jax import lax
from jax.experimental import pallas as pl
from jax.experimental.pallas import tpu as pltpu
from jax.experimental.pallas import tpu_sc as plsc

B = 1000
NPG = 100
N = B * NPG
D = 128
H = 128

G = 20
BP = 1024


def _sc_gather(fi_hbm, fv_hbm, idx_hbm, oi_hbm, ov_hbm,
               idx_v, rows_i, rows_v, sem_i, sem_v):
    nc = plsc.get_sparse_core_info().num_cores
    wid = lax.axis_index("s") * nc + lax.axis_index("c")
    bpw = BP // (nc * plsc.get_sparse_core_info().num_subcores)
    base = wid * bpw
    pltpu.sync_copy(idx_hbm.at[pl.ds(base, bpw)], idx_v)
    ci = pltpu.async_copy(fi_hbm.at[idx_v], rows_i, sem_i)
    cv = pltpu.async_copy(fv_hbm.at[idx_v], rows_v, sem_v)
    ci.wait()
    cv.wait()
    pltpu.sync_copy(rows_i, oi_hbm.at[pl.ds(base, bpw)])
    pltpu.sync_copy(rows_v, ov_hbm.at[pl.ds(base, bpw)])


def _gather_last_rows(feat_invar, feat_var, idx_pad):
    info = plsc.get_sparse_core_info()
    bpw = BP // (info.num_cores * info.num_subcores)
    mesh = plsc.VectorSubcoreMesh(core_axis_name="c", subcore_axis_name="s")
    k = functools.partial(
        pl.kernel, mesh=mesh,
        out_type=[jax.ShapeDtypeStruct((BP, D), jnp.float32),
                  jax.ShapeDtypeStruct((BP, D), jnp.float32)],
        scratch_types=[
            pltpu.VMEM((bpw,), jnp.int32),
            pltpu.VMEM((bpw, D), jnp.float32),
            pltpu.VMEM((bpw, D), jnp.float32),
            pltpu.SemaphoreType.DMA,
            pltpu.SemaphoreType.DMA,
        ],
    )(_sc_gather)
    return k(feat_invar, feat_var, idx_pad)


def _sigm(z):
    return 1.0 / (1.0 + jnp.exp(-z))


def _attn_body(fi_ref, fv_ref, gvi_ref, gvv_ref, wu_ref, bu_ref, wv_ref,
               we_ref, wet_ref, s2_ref, oi_ref, ov_ref):
    f32 = jnp.float32
    bf = jnp.bfloat16
    R = G * NPG
    dn = (((0,), (0,)), ((), ()))
    Xi_b = fi_ref[...].astype(bf)
    Xv_b = fv_ref[...].astype(bf)
    Wu_b = wu_ref[...].astype(bf)
    bu = bu_ref[...]
    Ui = jnp.dot(Xi_b, Wu_b, preferred_element_type=f32)
    Uv = jnp.dot(Xv_b, Wu_b, preferred_element_type=f32)
    Wv_b = wv_ref[...].astype(bf)
    Vi = jnp.dot(gvi_ref[0].astype(bf), Wv_b,
                 preferred_element_type=f32) + bu
    Vv = jnp.dot(gvv_ref[0].astype(bf), Wv_b,
                 preferred_element_type=f32) + bu
    S2 = s2_ref[...]
    S_b = S2[:, :G].astype(bf)
    Vbi = jnp.dot(S_b, Vi.astype(bf), preferred_element_type=f32)
    Vbv = jnp.dot(S_b, Vv.astype(bf), preferred_element_type=f32)
    m = jnp.sum(jnp.abs(wet_ref[...]))
    We_b = we_ref[...].astype(bf)
    e_ii = jnp.dot(_sigm(Ui + Vbi).astype(bf), We_b,
                   preferred_element_type=f32)
    e_vi = jnp.dot(_sigm(Uv + Vbi).astype(bf), We_b,
                   preferred_element_type=f32)
    e_iv = jnp.dot(_sigm(Ui + Vbv).astype(bf), We_b,
                   preferred_element_type=f32)
    e_vv = jnp.dot(_sigm(Uv + Vbv).astype(bf), We_b,
                   preferred_element_type=f32)
    x_ii = jnp.exp(e_ii - m)
    x_vi = jnp.exp(e_vi - m)
    x_iv = jnp.exp(e_iv - m)
    x_vv = jnp.exp(e_vv - m)
    lane = lax.broadcasted_iota(jnp.int32, (R, 2 * G), 1)
    half = lane < G
    A_i = (S2 * jnp.where(half, x_ii, x_iv)).astype(bf)
    A_v = (S2 * jnp.where(half, x_vi, x_vv)).astype(bf)
    Rp = (lax.dot_general(A_i, Xi_b, dn, preferred_element_type=f32) +
          lax.dot_general(A_v, Xv_b, dn, preferred_element_type=f32))
    lane2 = lax.broadcasted_iota(jnp.int32, (R, 2), 1)
    xs = jnp.where(lane2 < 1, x_ii + x_vi, x_iv + x_vv)
    sp = lax.dot_general(S2[:, :G], xs, dn,
                         preferred_element_type=f32)
    oi_ref[0] = Rp[:G] / sp[:, 0:1]
    ov_ref[0] = Rp[G:] / sp[:, 1:2]


def _attn_readout(fi, fv, gvi3, gvv3, Wu, bu2, Wv, We, WeT, S2):
    R = G * NPG
    return pl.pallas_call(
        _attn_body,
        grid=(B // G,),
        in_specs=[
            pl.BlockSpec((R, D), lambda i: (i, 0)),
            pl.BlockSpec((R, D), lambda i: (i, 0)),
            pl.BlockSpec((1, G, D), lambda i: (i, 0, 0)),
            pl.BlockSpec((1, G, D), lambda i: (i, 0, 0)),
            pl.BlockSpec((D, H), lambda i: (0, 0)),
            pl.BlockSpec((1, H), lambda i: (0, 0)),
            pl.BlockSpec((D, H), lambda i: (0, 0)),
            pl.BlockSpec((H, 1), lambda i: (0, 0)),
            pl.BlockSpec((1, H), lambda i: (0, 0)),
            pl.BlockSpec((R, 2 * G), lambda i: (0, 0)),
        ],
        out_specs=[pl.BlockSpec((1, G, D), lambda i: (i, 0, 0)),
                   pl.BlockSpec((1, G, D), lambda i: (i, 0, 0))],
        out_shape=[jax.ShapeDtypeStruct((B // G, G, D), jnp.float32),
                   jax.ShapeDtypeStruct((B // G, G, D), jnp.float32)],
    )(fi, fv, gvi3, gvv3, Wu, bu2, Wv, We, WeT, S2)


def kernel(feat_invar, feat_var, last_nodes, Wu, bu, Wv, We):
    idx = last_nodes.astype(jnp.int32)
    idx_pad = jnp.concatenate([idx, jnp.zeros((BP - B,), jnp.int32)])
    gi, gv = _gather_last_rows(feat_invar, feat_var, idx_pad)
    bu2 = bu.reshape(1, H)
    WeT = We.reshape(1, H)
    S = jnp.repeat(jnp.eye(G, dtype=jnp.float32), NPG, axis=0)
    S2 = jnp.concatenate([S, S], axis=1)
    ri, rv = _attn_readout(feat_invar, feat_var,
                           gi[:B].reshape(B // G, G, D),
                           gv[:B].reshape(B // G, G, D),
                           Wu, bu2, Wv, We, WeT, S2)
    return (ri.reshape(B, D)[:, None, :], rv.reshape(B, D)[:, None, :])


# speedup vs baseline: 35.1344x; 1.1942x over previous
import functools

import jax
import jax.numpy as jnp
from jax import lax
from jax.experimental import pallas as pl
from jax.experimental.pallas import tpu as pltpu
from jax.experimental.pallas import tpu_sc as plsc

B = 1000
NPG = 100
N = B * NPG
D = 128
H = 128

G = 20
BP = 1280


def _sc_gather(fi_hbm, fv_hbm, idx_hbm, oi_hbm, ov_hbm,
               idx_v, rows_i, rows_v, sem_i, sem_v):
    nc = plsc.get_sparse_core_info().num_cores
    wid = lax.axis_index("s") * nc + lax.axis_index("c")
    bpw = BP // (nc * plsc.get_sparse_core_info().num_subcores)
    base = wid * bpw
    pltpu.sync_copy(idx_hbm.at[pl.ds(base, bpw)], idx_v)
    ci = pltpu.async_copy(fi_hbm.at[idx_v], rows_i, sem_i)
    cv = pltpu.async_copy(fv_hbm.at[idx_v], rows_v, sem_v)
    ci.wait()
    cv.wait()
    pltpu.sync_copy(rows_i, oi_hbm.at[pl.ds(base, bpw)])
    pltpu.sync_copy(rows_v, ov_hbm.at[pl.ds(base, bpw)])


def _gather_last_rows(feat_invar, feat_var, idx_pad):
    info = plsc.get_sparse_core_info()
    bpw = BP // (info.num_cores * info.num_subcores)
    mesh = plsc.VectorSubcoreMesh(core_axis_name="c", subcore_axis_name="s")
    k = functools.partial(
        pl.kernel, mesh=mesh,
        out_type=[jax.ShapeDtypeStruct((BP, D), jnp.float32),
                  jax.ShapeDtypeStruct((BP, D), jnp.float32)],
        scratch_types=[
            pltpu.VMEM((bpw,), jnp.int32),
            pltpu.VMEM((bpw, D), jnp.float32),
            pltpu.VMEM((bpw, D), jnp.float32),
            pltpu.SemaphoreType.DMA,
            pltpu.SemaphoreType.DMA,
        ],
    )(_sc_gather)
    return k(feat_invar, feat_var, idx_pad)


def _sigm(z):
    return 1.0 / (1.0 + jnp.exp(-z))


def _attn_body(fi_ref, fv_ref, gvi_ref, gvv_ref, wu_ref, bu_ref, wv_ref,
               wel_ref, wer_ref, wet_ref, s2_ref, oi_ref, ov_ref):
    f32 = jnp.float32
    bf = jnp.bfloat16
    R = G * NPG
    dn = (((0,), (0,)), ((), ()))
    Xi_b = fi_ref[...].astype(bf)
    Xv_b = fv_ref[...].astype(bf)
    Wu_b = wu_ref[...].astype(bf)
    bu = bu_ref[...]
    Ui = jnp.dot(Xi_b, Wu_b, preferred_element_type=f32)
    Uv = jnp.dot(Xv_b, Wu_b, preferred_element_type=f32)
    Wv_b = wv_ref[...].astype(bf)
    Vi = jnp.dot(gvi_ref[0].astype(bf), Wv_b,
                 preferred_element_type=f32) + bu
    Vv = jnp.dot(gvv_ref[0].astype(bf), Wv_b,
                 preferred_element_type=f32) + bu
    S2 = s2_ref[...]
    S_b = S2[:, :G].astype(bf)
    Vbi = jnp.dot(S_b, Vi.astype(bf), preferred_element_type=f32)
    Vbv = jnp.dot(S_b, Vv.astype(bf), preferred_element_type=f32)
    m = jnp.sum(jnp.abs(wet_ref[...]))
    WeL = wel_ref[...].astype(bf)
    WeR = wer_ref[...].astype(bf)
    T_i = (jnp.dot(_sigm(Ui + Vbi).astype(bf), WeL,
                   preferred_element_type=f32) +
           jnp.dot(_sigm(Ui + Vbv).astype(bf), WeR,
                   preferred_element_type=f32))
    T_v = (jnp.dot(_sigm(Uv + Vbi).astype(bf), WeL,
                   preferred_element_type=f32) +
           jnp.dot(_sigm(Uv + Vbv).astype(bf), WeR,
                   preferred_element_type=f32))
    A_i = S2 * jnp.exp(T_i - m)
    A_v = S2 * jnp.exp(T_v - m)
    A_i_b = A_i.astype(bf)
    A_v_b = A_v.astype(bf)
    Rp = (lax.dot_general(A_i_b, Xi_b, dn, preferred_element_type=f32) +
          lax.dot_general(A_v_b, Xv_b, dn, preferred_element_type=f32))
    ones_b = jnp.ones((R, 1), bf)
    sp = lax.dot_general((A_i + A_v).astype(bf), ones_b, dn,
                         preferred_element_type=f32)
    oi_ref[0] = Rp[:G] / sp[:G]
    ov_ref[0] = Rp[G:] / sp[G:]


def _attn_readout(fi, fv, gvi3, gvv3, Wu, bu2, Wv, WeL, WeR, WeT, S2):
    R = G * NPG
    return pl.pallas_call(
        _attn_body,
        grid=(B // G,),
        in_specs=[
            pl.BlockSpec((R, D), lambda i: (i, 0)),
            pl.BlockSpec((R, D), lambda i: (i, 0)),
            pl.BlockSpec((1, G, D), lambda i: (i, 0, 0)),
            pl.BlockSpec((1, G, D), lambda i: (i, 0, 0)),
            pl.BlockSpec((D, H), lambda i: (0, 0)),
            pl.BlockSpec((1, H), lambda i: (0, 0)),
            pl.BlockSpec((D, H), lambda i: (0, 0)),
            pl.BlockSpec((H, 2 * G), lambda i: (0, 0)),
            pl.BlockSpec((H, 2 * G), lambda i: (0, 0)),
            pl.BlockSpec((1, H), lambda i: (0, 0)),
            pl.BlockSpec((R, 2 * G), lambda i: (0, 0)),
        ],
        out_specs=[pl.BlockSpec((1, G, D), lambda i: (i, 0, 0)),
                   pl.BlockSpec((1, G, D), lambda i: (i, 0, 0))],
        out_shape=[jax.ShapeDtypeStruct((B // G, G, D), jnp.float32),
                   jax.ShapeDtypeStruct((B // G, G, D), jnp.float32)],
    )(fi, fv, gvi3, gvv3, Wu, bu2, Wv, WeL, WeR, WeT, S2)


def kernel(feat_invar, feat_var, last_nodes, Wu, bu, Wv, We):
    idx = last_nodes.astype(jnp.int32)
    idx_pad = jnp.concatenate([idx, jnp.zeros((BP - B,), jnp.int32)])
    gi, gv = _gather_last_rows(feat_invar, feat_var, idx_pad)
    bu2 = bu.reshape(1, H)
    WeT = We.reshape(1, H)
    zeros = jnp.zeros((H, G), jnp.float32)
    WeL = jnp.concatenate([jnp.tile(We, (1, G)), zeros], axis=1)
    WeR = jnp.concatenate([zeros, jnp.tile(We, (1, G))], axis=1)
    S = jnp.repeat(jnp.eye(G, dtype=jnp.float32), NPG, axis=0)
    S2 = jnp.concatenate([S, S], axis=1)
    ri, rv = _attn_readout(feat_invar, feat_var,
                           gi.reshape(BP // G, G, D),
                           gv.reshape(BP // G, G, D),
                           Wu, bu2, Wv, WeL, WeR, WeT, S2)
    return (ri.reshape(B, D)[:, None, :], rv.reshape(B, D)[:, None, :])


# speedup vs baseline: 37.1893x; 1.0585x over previous
import functools

import jax
import jax.numpy as jnp
from jax import lax
from jax.experimental import pallas as pl
from jax.experimental.pallas import tpu as pltpu
from jax.experimental.pallas import tpu_sc as plsc

B = 1000
NPG = 100
N = B * NPG
D = 128
H = 128

G = 20
BP = 1024


def _sc_gather(fi_hbm, fv_hbm, idx_hbm, oi_hbm, ov_hbm,
               idx_v, rows_i, rows_v, sem_i, sem_v):
    nc = plsc.get_sparse_core_info().num_cores
    wid = lax.axis_index("s") * nc + lax.axis_index("c")
    bpw = BP // (nc * plsc.get_sparse_core_info().num_subcores)
    base = wid * bpw
    pltpu.sync_copy(idx_hbm.at[pl.ds(base, bpw)], idx_v)
    ci = pltpu.async_copy(fi_hbm.at[idx_v], rows_i, sem_i)
    cv = pltpu.async_copy(fv_hbm.at[idx_v], rows_v, sem_v)
    ci.wait()
    cv.wait()
    pltpu.sync_copy(rows_i, oi_hbm.at[pl.ds(base, bpw)])
    pltpu.sync_copy(rows_v, ov_hbm.at[pl.ds(base, bpw)])


def _gather_last_rows(feat_invar, feat_var, idx_pad):
    info = plsc.get_sparse_core_info()
    bpw = BP // (info.num_cores * info.num_subcores)
    mesh = plsc.VectorSubcoreMesh(core_axis_name="c", subcore_axis_name="s")
    k = functools.partial(
        pl.kernel, mesh=mesh,
        out_type=[jax.ShapeDtypeStruct((BP, D), jnp.float32),
                  jax.ShapeDtypeStruct((BP, D), jnp.float32)],
        scratch_types=[
            pltpu.VMEM((bpw,), jnp.int32),
            pltpu.VMEM((bpw, D), jnp.float32),
            pltpu.VMEM((bpw, D), jnp.float32),
            pltpu.SemaphoreType.DMA,
            pltpu.SemaphoreType.DMA,
        ],
    )(_sc_gather)
    return k(feat_invar, feat_var, idx_pad)


def _sigm(z):
    return 1.0 / (1.0 + jnp.exp(-z))


def _attn_body(fi_ref, fv_ref, gvi_ref, gvv_ref, wu_ref, bu_ref, wv_ref,
               wel_ref, wer_ref, wet_ref, s2_ref, oi_ref, ov_ref):
    f32 = jnp.float32
    bf = jnp.bfloat16
    R = G * NPG
    dn = (((0,), (0,)), ((), ()))
    Xi_b = fi_ref[...].astype(bf)
    Xv_b = fv_ref[...].astype(bf)
    Wu_b = wu_ref[...].astype(bf)
    bu = bu_ref[...]
    Ui = jnp.dot(Xi_b, Wu_b, preferred_element_type=f32)
    Uv = jnp.dot(Xv_b, Wu_b, preferred_element_type=f32)
    Wv_b = wv_ref[...].astype(bf)
    Vi = jnp.dot(gvi_ref[0].astype(bf), Wv_b,
                 preferred_element_type=f32) + bu
    Vv = jnp.dot(gvv_ref[0].astype(bf), Wv_b,
                 preferred_element_type=f32) + bu
    S2 = s2_ref[...]
    S_b = S2[:, :G].astype(bf)
    Vbi = jnp.dot(S_b, Vi.astype(bf), preferred_element_type=f32)
    Vbv = jnp.dot(S_b, Vv.astype(bf), preferred_element_type=f32)
    m = jnp.sum(jnp.abs(wet_ref[...]))
    WeL = wel_ref[...].astype(bf)
    WeR = wer_ref[...].astype(bf)
    T_i = (jnp.dot(_sigm(Ui + Vbi).astype(bf), WeL,
                   preferred_element_type=f32) +
           jnp.dot(_sigm(Ui + Vbv).astype(bf), WeR,
                   preferred_element_type=f32))
    T_v = (jnp.dot(_sigm(Uv + Vbi).astype(bf), WeL,
                   preferred_element_type=f32) +
           jnp.dot(_sigm(Uv + Vbv).astype(bf), WeR,
                   preferred_element_type=f32))
    A_i = S2 * jnp.exp(T_i - m)
    A_v = S2 * jnp.exp(T_v - m)
    A_i_b = A_i.astype(bf)
    A_v_b = A_v.astype(bf)
    Rp = (lax.dot_general(A_i_b, Xi_b, dn, preferred_element_type=f32) +
          lax.dot_general(A_v_b, Xv_b, dn, preferred_element_type=f32))
    ones_b = jnp.ones((R, 1), bf)
    sp = lax.dot_general((A_i + A_v).astype(bf), ones_b, dn,
                         preferred_element_type=f32)
    oi_ref[0] = Rp[:G] / sp[:G]
    ov_ref[0] = Rp[G:] / sp[G:]


def _attn_readout(fi, fv, gvi3, gvv3, Wu, bu2, Wv, WeL, WeR, WeT, S2):
    R = G * NPG
    return pl.pallas_call(
        _attn_body,
        grid=(B // G,),
        in_specs=[
            pl.BlockSpec((R, D), lambda i: (i, 0)),
            pl.BlockSpec((R, D), lambda i: (i, 0)),
            pl.BlockSpec((1, G, D), lambda i: (i, 0, 0)),
            pl.BlockSpec((1, G, D), lambda i: (i, 0, 0)),
            pl.BlockSpec((D, H), lambda i: (0, 0)),
            pl.BlockSpec((1, H), lambda i: (0, 0)),
            pl.BlockSpec((D, H), lambda i: (0, 0)),
            pl.BlockSpec((H, 2 * G), lambda i: (0, 0)),
            pl.BlockSpec((H, 2 * G), lambda i: (0, 0)),
            pl.BlockSpec((1, H), lambda i: (0, 0)),
            pl.BlockSpec((R, 2 * G), lambda i: (0, 0)),
        ],
        out_specs=[pl.BlockSpec((1, G, D), lambda i: (i, 0, 0)),
                   pl.BlockSpec((1, G, D), lambda i: (i, 0, 0))],
        out_shape=[jax.ShapeDtypeStruct((B // G, G, D), jnp.float32),
                   jax.ShapeDtypeStruct((B // G, G, D), jnp.float32)],
    )(fi, fv, gvi3, gvv3, Wu, bu2, Wv, WeL, WeR, WeT, S2)


def kernel(feat_invar, feat_var, last_nodes, Wu, bu, Wv, We):
    idx = last_nodes.astype(jnp.int32)
    idx_pad = jnp.concatenate([idx, jnp.zeros((BP - B,), jnp.int32)])
    gi, gv = _gather_last_rows(feat_invar, feat_var, idx_pad)
    bu2 = bu.reshape(1, H)
    WeT = We.reshape(1, H)
    zeros = jnp.zeros((H, G), jnp.float32)
    WeL = jnp.concatenate([jnp.tile(We, (1, G)), zeros], axis=1)
    WeR = jnp.concatenate([zeros, jnp.tile(We, (1, G))], axis=1)
    S = jnp.repeat(jnp.eye(G, dtype=jnp.float32), NPG, axis=0)
    S2 = jnp.concatenate([S, S], axis=1)
    ri, rv = _attn_readout(feat_invar, feat_var,
                           gi[:B].reshape(B // G, G, D),
                           gv[:B].reshape(B // G, G, D),
                           Wu, bu2, Wv, WeL, WeR, WeT, S2)
    return (ri.reshape(B, D)[:, None, :], rv.reshape(B, D)[:, None, :])


# speedup vs baseline: 48.7686x; 1.3114x over previous
import functools

import jax
import jax.numpy as jnp
from jax import lax
from jax.experimental import pallas as pl
from jax.experimental.pallas import tpu as pltpu
from jax.experimental.pallas import tpu_sc as plsc

B = 1000
NPG = 100
N = B * NPG
D = 128
H = 128

G = 40
BP = 1024


def _sc_gather(fi_hbm, fv_hbm, idx_hbm, oi_hbm, ov_hbm,
               idx_v, rows_i, rows_v, sem_i, sem_v):
    nc = plsc.get_sparse_core_info().num_cores
    wid = lax.axis_index("s") * nc + lax.axis_index("c")
    bpw = BP // (nc * plsc.get_sparse_core_info().num_subcores)
    base = wid * bpw
    pltpu.sync_copy(idx_hbm.at[pl.ds(base, bpw)], idx_v)
    ci = pltpu.async_copy(fi_hbm.at[idx_v], rows_i, sem_i)
    cv = pltpu.async_copy(fv_hbm.at[idx_v], rows_v, sem_v)
    ci.wait()
    cv.wait()
    pltpu.sync_copy(rows_i, oi_hbm.at[pl.ds(base, bpw)])
    pltpu.sync_copy(rows_v, ov_hbm.at[pl.ds(base, bpw)])


def _gather_last_rows(feat_invar, feat_var, idx_pad):
    info = plsc.get_sparse_core_info()
    bpw = BP // (info.num_cores * info.num_subcores)
    mesh = plsc.VectorSubcoreMesh(core_axis_name="c", subcore_axis_name="s")
    k = functools.partial(
        pl.kernel, mesh=mesh,
        out_type=[jax.ShapeDtypeStruct((BP, D), jnp.float32),
                  jax.ShapeDtypeStruct((BP, D), jnp.float32)],
        scratch_types=[
            pltpu.VMEM((bpw,), jnp.int32),
            pltpu.VMEM((bpw, D), jnp.float32),
            pltpu.VMEM((bpw, D), jnp.float32),
            pltpu.SemaphoreType.DMA,
            pltpu.SemaphoreType.DMA,
        ],
    )(_sc_gather)
    return k(feat_invar, feat_var, idx_pad)


_NLOG2E = -1.4426950408889634


def _sigm(z):
    return 1.0 / (1.0 + jnp.exp2(z * _NLOG2E))


def _attn_body(fi_ref, fv_ref, gvi_ref, gvv_ref, wu_ref, bu_ref, wv_ref,
               welr_ref, wet_ref, s2_ref, oi_ref, ov_ref):
    f32 = jnp.float32
    bf = jnp.bfloat16
    R = G * NPG
    dn = (((0,), (0,)), ((), ()))
    Xi_b = fi_ref[...].astype(bf)
    Xv_b = fv_ref[...].astype(bf)
    Wu_b = wu_ref[...].astype(bf)
    bu = bu_ref[...]
    Ui = jnp.dot(Xi_b, Wu_b, preferred_element_type=f32)
    Uv = jnp.dot(Xv_b, Wu_b, preferred_element_type=f32)
    Wv_b = wv_ref[...].astype(bf)
    Vi = jnp.dot(gvi_ref[0].astype(bf), Wv_b,
                 preferred_element_type=f32) + bu
    Vv = jnp.dot(gvv_ref[0].astype(bf), Wv_b,
                 preferred_element_type=f32) + bu
    S2 = s2_ref[...]
    S_b = S2[:, :G].astype(bf)
    Vbi = jnp.dot(S_b, Vi.astype(bf), preferred_element_type=f32)
    Vbv = jnp.dot(S_b, Vv.astype(bf), preferred_element_type=f32)
    m = jnp.sum(jnp.abs(wet_ref[...]))
    WeLR = welr_ref[...].astype(bf)
    T_i = jnp.dot(
        jnp.concatenate([_sigm(Ui + Vbi).astype(bf),
                         _sigm(Ui + Vbv).astype(bf)], axis=1),
        WeLR, preferred_element_type=f32)
    T_v = jnp.dot(
        jnp.concatenate([_sigm(Uv + Vbi).astype(bf),
                         _sigm(Uv + Vbv).astype(bf)], axis=1),
        WeLR, preferred_element_type=f32)
    A_i = S2 * jnp.exp(T_i - m)
    A_v = S2 * jnp.exp(T_v - m)
    A_i_b = A_i.astype(bf)
    A_v_b = A_v.astype(bf)
    Rp = (lax.dot_general(A_i_b, Xi_b, dn, preferred_element_type=f32) +
          lax.dot_general(A_v_b, Xv_b, dn, preferred_element_type=f32))
    ones_b = jnp.ones((R, 1), bf)
    sp = lax.dot_general((A_i + A_v).astype(bf), ones_b, dn,
                         preferred_element_type=f32)
    oi_ref[0] = Rp[:G] / sp[:G]
    ov_ref[0] = Rp[G:] / sp[G:]


def _attn_readout(fi, fv, gvi3, gvv3, Wu, bu2, Wv, WeLR, WeT, S2):
    R = G * NPG
    return pl.pallas_call(
        _attn_body,
        grid=(B // G,),
        in_specs=[
            pl.BlockSpec((R, D), lambda i: (i, 0)),
            pl.BlockSpec((R, D), lambda i: (i, 0)),
            pl.BlockSpec((1, G, D), lambda i: (i, 0, 0)),
            pl.BlockSpec((1, G, D), lambda i: (i, 0, 0)),
            pl.BlockSpec((D, H), lambda i: (0, 0)),
            pl.BlockSpec((1, H), lambda i: (0, 0)),
            pl.BlockSpec((D, H), lambda i: (0, 0)),
            pl.BlockSpec((2 * H, 2 * G), lambda i: (0, 0)),
            pl.BlockSpec((1, H), lambda i: (0, 0)),
            pl.BlockSpec((R, 2 * G), lambda i: (0, 0)),
        ],
        out_specs=[pl.BlockSpec((1, G, D), lambda i: (i, 0, 0)),
                   pl.BlockSpec((1, G, D), lambda i: (i, 0, 0))],
        out_shape=[jax.ShapeDtypeStruct((B // G, G, D), jnp.float32),
                   jax.ShapeDtypeStruct((B // G, G, D), jnp.float32)],
    )(fi, fv, gvi3, gvv3, Wu, bu2, Wv, WeLR, WeT, S2)


def kernel(feat_invar, feat_var, last_nodes, Wu, bu, Wv, We):
    idx = last_nodes.astype(jnp.int32)
    idx_pad = jnp.concatenate([idx, jnp.zeros((BP - B,), jnp.int32)])
    gi, gv = _gather_last_rows(feat_invar, feat_var, idx_pad)
    bu2 = bu.reshape(1, H)
    WeT = We.reshape(1, H)
    zeros = jnp.zeros((H, G), jnp.float32)
    tiled = jnp.tile(We, (1, G))
    WeLR = jnp.concatenate([
        jnp.concatenate([tiled, zeros], axis=1),
        jnp.concatenate([zeros, tiled], axis=1),
    ], axis=0)
    S = jnp.repeat(jnp.eye(G, dtype=jnp.float32), NPG, axis=0)
    S2 = jnp.concatenate([S, S], axis=1)
    ri, rv = _attn_readout(feat_invar, feat_var,
                           gi[:B].reshape(B // G, G, D),
                           gv[:B].reshape(B // G, G, D),
                           Wu, bu2, Wv, WeLR, WeT, S2)
    return (ri.reshape(B, D)[:, None, :], rv.reshape(B, D)[:, None, :])


# speedup vs baseline: 56.4491x; 1.1575x over previous
import functools

import jax
import jax.numpy as jnp
from jax import lax
from jax.experimental import pallas as pl
from jax.experimental.pallas import tpu as pltpu
from jax.experimental.pallas import tpu_sc as plsc

B = 1000
NPG = 100
N = B * NPG
D = 128
H = 128

G = 40
BP = 1024


def _sc_gather(fi_hbm, fv_hbm, idx_hbm, oi_hbm, ov_hbm,
               idx_v, rows_i, rows_v, sem_i, sem_v):
    nc = plsc.get_sparse_core_info().num_cores
    wid = lax.axis_index("s") * nc + lax.axis_index("c")
    bpw = BP // (nc * plsc.get_sparse_core_info().num_subcores)
    base = wid * bpw
    pltpu.sync_copy(idx_hbm.at[pl.ds(base, bpw)], idx_v)
    ci = pltpu.async_copy(fi_hbm.at[idx_v], rows_i, sem_i)
    cv = pltpu.async_copy(fv_hbm.at[idx_v], rows_v, sem_v)
    ci.wait()
    cv.wait()
    pltpu.sync_copy(rows_i, oi_hbm.at[pl.ds(base, bpw)])
    pltpu.sync_copy(rows_v, ov_hbm.at[pl.ds(base, bpw)])


def _gather_last_rows(feat_invar, feat_var, idx_pad):
    info = plsc.get_sparse_core_info()
    bpw = BP // (info.num_cores * info.num_subcores)
    mesh = plsc.VectorSubcoreMesh(core_axis_name="c", subcore_axis_name="s")
    k = functools.partial(
        pl.kernel, mesh=mesh,
        out_type=[jax.ShapeDtypeStruct((BP, D), jnp.float32),
                  jax.ShapeDtypeStruct((BP, D), jnp.float32)],
        scratch_types=[
            pltpu.VMEM((bpw,), jnp.int32),
            pltpu.VMEM((bpw, D), jnp.float32),
            pltpu.VMEM((bpw, D), jnp.float32),
            pltpu.SemaphoreType.DMA,
            pltpu.SemaphoreType.DMA,
        ],
    )(_sc_gather)
    return k(feat_invar, feat_var, idx_pad)


_LOG2E = 1.4426950408889634


def _attn_body(fi_ref, fv_ref, gvi_ref, gvv_ref, wu_ref, bu_ref, wv_ref,
               welr_ref, wet_ref, slog_ref, sbf_ref, oi_ref, ov_ref):
    f32 = jnp.float32
    bf = jnp.bfloat16
    dn = (((0,), (0,)), ((), ()))
    Xi_b = fi_ref[...].astype(bf)
    Xv_b = fv_ref[...].astype(bf)
    Wu_b = wu_ref[...].astype(bf)
    bu = bu_ref[...]
    Ui = jnp.dot(Xi_b, Wu_b, preferred_element_type=f32)
    Uv = jnp.dot(Xv_b, Wu_b, preferred_element_type=f32)
    Wv_b = wv_ref[...].astype(bf)
    Vi = jnp.dot(gvi_ref[0].astype(bf), Wv_b,
                 preferred_element_type=f32) + bu
    Vv = jnp.dot(gvv_ref[0].astype(bf), Wv_b,
                 preferred_element_type=f32) + bu
    S_b = sbf_ref[...]
    Vbi = jnp.dot(S_b, Vi.astype(bf), preferred_element_type=f32)
    Vbv = jnp.dot(S_b, Vv.astype(bf), preferred_element_type=f32)
    m2 = jnp.sum(jnp.abs(wet_ref[...]))
    WeLR = welr_ref[...].astype(bf)
    T_i = jnp.dot(
        jnp.concatenate([jnp.tanh(Ui + Vbi).astype(bf),
                         jnp.tanh(Ui + Vbv).astype(bf)], axis=1),
        WeLR, preferred_element_type=f32)
    T_v = jnp.dot(
        jnp.concatenate([jnp.tanh(Uv + Vbi).astype(bf),
                         jnp.tanh(Uv + Vbv).astype(bf)], axis=1),
        WeLR, preferred_element_type=f32)
    c2 = slog_ref[...] - m2
    A_i_b = jnp.exp2(T_i + c2).astype(bf)
    A_v_b = jnp.exp2(T_v + c2).astype(bf)
    Rp = (lax.dot_general(A_i_b, Xi_b, dn, preferred_element_type=f32) +
          lax.dot_general(A_v_b, Xv_b, dn, preferred_element_type=f32))
    ones_b = jnp.ones((G * NPG, 1), bf)
    sp = (lax.dot_general(A_i_b, ones_b, dn, preferred_element_type=f32) +
          lax.dot_general(A_v_b, ones_b, dn, preferred_element_type=f32))
    oi_ref[0] = Rp[:G] / sp[:G]
    ov_ref[0] = Rp[G:] / sp[G:]


def _attn_readout(fi, fv, gvi3, gvv3, Wu2, bu2, Wv2, WeLR, WeT, SLOG, SBF):
    R = G * NPG
    return pl.pallas_call(
        _attn_body,
        grid=(B // G,),
        in_specs=[
            pl.BlockSpec((R, D), lambda i: (i, 0)),
            pl.BlockSpec((R, D), lambda i: (i, 0)),
            pl.BlockSpec((1, G, D), lambda i: (i, 0, 0)),
            pl.BlockSpec((1, G, D), lambda i: (i, 0, 0)),
            pl.BlockSpec((D, H), lambda i: (0, 0)),
            pl.BlockSpec((1, H), lambda i: (0, 0)),
            pl.BlockSpec((D, H), lambda i: (0, 0)),
            pl.BlockSpec((2 * H, 2 * G), lambda i: (0, 0)),
            pl.BlockSpec((1, H), lambda i: (0, 0)),
            pl.BlockSpec((R, 2 * G), lambda i: (0, 0)),
            pl.BlockSpec((R, G), lambda i: (0, 0)),
        ],
        out_specs=[pl.BlockSpec((1, G, D), lambda i: (i, 0, 0)),
                   pl.BlockSpec((1, G, D), lambda i: (i, 0, 0))],
        out_shape=[jax.ShapeDtypeStruct((B // G, G, D), jnp.float32),
                   jax.ShapeDtypeStruct((B // G, G, D), jnp.float32)],
    )(fi, fv, gvi3, gvv3, Wu2, bu2, Wv2, WeLR, WeT, SLOG, SBF)


def kernel(feat_invar, feat_var, last_nodes, Wu, bu, Wv, We):
    idx = last_nodes.astype(jnp.int32)
    idx_pad = jnp.concatenate([idx, jnp.zeros((BP - B,), jnp.int32)])
    gi, gv = _gather_last_rows(feat_invar, feat_var, idx_pad)
    Wu2 = Wu * 0.5
    Wv2 = Wv * 0.5
    bu2 = bu.reshape(1, H) * 0.5
    Wes = We * (0.5 * _LOG2E)
    WeT = jnp.abs(Wes).reshape(1, H)
    zeros = jnp.zeros((H, G), jnp.float32)
    tiled = jnp.tile(Wes, (1, G))
    WeLR = jnp.concatenate([
        jnp.concatenate([tiled, zeros], axis=1),
        jnp.concatenate([zeros, tiled], axis=1),
    ], axis=0)
    S = jnp.repeat(jnp.eye(G, dtype=jnp.float32), NPG, axis=0)
    SLOG = (jnp.concatenate([S, S], axis=1) - 1.0) * 100.0
    SBF = S.astype(jnp.bfloat16)
    ri, rv = _attn_readout(feat_invar, feat_var,
                           gi[:B].reshape(B // G, G, D),
                           gv[:B].reshape(B // G, G, D),
                           Wu2, bu2, Wv2, WeLR, WeT, SLOG, SBF)
    return (ri.reshape(B, D)[:, None, :], rv.reshape(B, D)[:, None, :])


# speedup vs baseline: 57.0036x; 1.0098x over previous
import functools

import jax
import jax.numpy as jnp
from jax import lax
from jax.experimental import pallas as pl
from jax.experimental.pallas import tpu as pltpu
from jax.experimental.pallas import tpu_sc as plsc

B = 1000
NPG = 100
N = B * NPG
D = 128
H = 128

G = 40
BP = 1024


def _sc_gather(fi_hbm, fv_hbm, idx_hbm, oi_hbm, ov_hbm,
               idx_v, rows_i, rows_v, sem_i, sem_v):
    nc = plsc.get_sparse_core_info().num_cores
    wid = lax.axis_index("s") * nc + lax.axis_index("c")
    bpw = BP // (nc * plsc.get_sparse_core_info().num_subcores)
    base = wid * bpw
    pltpu.sync_copy(idx_hbm.at[pl.ds(base, bpw)], idx_v)
    ci = pltpu.async_copy(fi_hbm.at[idx_v], rows_i, sem_i)
    cv = pltpu.async_copy(fv_hbm.at[idx_v], rows_v, sem_v)
    ci.wait()
    cv.wait()
    pltpu.sync_copy(rows_i, oi_hbm.at[pl.ds(base, bpw)])
    pltpu.sync_copy(rows_v, ov_hbm.at[pl.ds(base, bpw)])


def _gather_last_rows(feat_invar, feat_var, idx_pad):
    info = plsc.get_sparse_core_info()
    bpw = BP // (info.num_cores * info.num_subcores)
    mesh = plsc.VectorSubcoreMesh(core_axis_name="c", subcore_axis_name="s")
    k = functools.partial(
        pl.kernel, mesh=mesh,
        out_type=[jax.ShapeDtypeStruct((BP, D), jnp.float32),
                  jax.ShapeDtypeStruct((BP, D), jnp.float32)],
        scratch_types=[
            pltpu.VMEM((bpw,), jnp.int32),
            pltpu.VMEM((bpw, D), jnp.float32),
            pltpu.VMEM((bpw, D), jnp.float32),
            pltpu.SemaphoreType.DMA,
            pltpu.SemaphoreType.DMA,
        ],
    )(_sc_gather)
    return k(feat_invar, feat_var, idx_pad)


_LOG2E = 1.4426950408889634


def _attn_body(fi_ref, fv_ref, gvi_ref, gvv_ref, wu_ref, bu_ref, wv_ref,
               we_ref, slog_ref, sbf_ref, oi_ref, ov_ref):
    f32 = jnp.float32
    bf = jnp.bfloat16
    dn = (((0,), (0,)), ((), ()))
    Xi_b = fi_ref[...].astype(bf)
    Xv_b = fv_ref[...].astype(bf)
    Wu_b = (wu_ref[...] * 0.5).astype(bf)
    bu = bu_ref[...] * 0.5
    Ui = jnp.dot(Xi_b, Wu_b, preferred_element_type=f32)
    Uv = jnp.dot(Xv_b, Wu_b, preferred_element_type=f32)
    Wv_b = (wv_ref[...] * 0.5).astype(bf)
    Vi = jnp.dot(gvi_ref[...].astype(bf), Wv_b,
                 preferred_element_type=f32) + bu
    Vv = jnp.dot(gvv_ref[...].astype(bf), Wv_b,
                 preferred_element_type=f32) + bu
    S_b = sbf_ref[...]
    Vbi = jnp.dot(S_b, Vi.astype(bf), preferred_element_type=f32)
    Vbv = jnp.dot(S_b, Vv.astype(bf), preferred_element_type=f32)
    We_s = we_ref[...] * (0.5 * _LOG2E)
    lane = lax.broadcasted_iota(jnp.int32, (1, 2 * G), 1)
    onesL = (lane < G).astype(f32)
    onesR = 1.0 - onesL
    WeLR = jnp.concatenate([jnp.dot(We_s, onesL), jnp.dot(We_s, onesR)],
                           axis=0).astype(bf)
    T_i = jnp.dot(
        jnp.concatenate([jnp.tanh(Ui + Vbi).astype(bf),
                         jnp.tanh(Ui + Vbv).astype(bf)], axis=1),
        WeLR, preferred_element_type=f32)
    T_v = jnp.dot(
        jnp.concatenate([jnp.tanh(Uv + Vbi).astype(bf),
                         jnp.tanh(Uv + Vbv).astype(bf)], axis=1),
        WeLR, preferred_element_type=f32)
    slog = slog_ref[...]
    A_i_b = jnp.exp2(T_i + slog).astype(bf)
    A_v_b = jnp.exp2(T_v + slog).astype(bf)
    Rp = (lax.dot_general(A_i_b, Xi_b, dn, preferred_element_type=f32) +
          lax.dot_general(A_v_b, Xv_b, dn, preferred_element_type=f32))
    ones_b = jnp.ones((G * NPG, 1), bf)
    sp = (lax.dot_general(A_i_b, ones_b, dn, preferred_element_type=f32) +
          lax.dot_general(A_v_b, ones_b, dn, preferred_element_type=f32))
    oi_ref[0] = Rp[:G] / sp[:G]
    ov_ref[0] = Rp[G:] / sp[G:]


def _attn_readout(fi, fv, gi, gv, Wu, bu2, Wv, We, SLOG, SBF):
    R = G * NPG
    return pl.pallas_call(
        _attn_body,
        grid=(B // G,),
        in_specs=[
            pl.BlockSpec((R, D), lambda i: (i, 0)),
            pl.BlockSpec((R, D), lambda i: (i, 0)),
            pl.BlockSpec((G, D), lambda i: (i, 0)),
            pl.BlockSpec((G, D), lambda i: (i, 0)),
            pl.BlockSpec((D, H), lambda i: (0, 0)),
            pl.BlockSpec((1, H), lambda i: (0, 0)),
            pl.BlockSpec((D, H), lambda i: (0, 0)),
            pl.BlockSpec((H, 1), lambda i: (0, 0)),
            pl.BlockSpec((R, 2 * G), lambda i: (0, 0)),
            pl.BlockSpec((R, G), lambda i: (0, 0)),
        ],
        out_specs=[pl.BlockSpec((1, G, D), lambda i: (i, 0, 0)),
                   pl.BlockSpec((1, G, D), lambda i: (i, 0, 0))],
        out_shape=[jax.ShapeDtypeStruct((B // G, G, D), jnp.float32),
                   jax.ShapeDtypeStruct((B // G, G, D), jnp.float32)],
    )(fi, fv, gi, gv, Wu, bu2, Wv, We, SLOG, SBF)


def kernel(feat_invar, feat_var, last_nodes, Wu, bu, Wv, We):
    idx = last_nodes.astype(jnp.int32)
    idx_pad = jnp.concatenate([idx, jnp.zeros((BP - B,), jnp.int32)])
    gi, gv = _gather_last_rows(feat_invar, feat_var, idx_pad)
    bu2 = bu.reshape(1, H)
    S = jnp.repeat(jnp.eye(G, dtype=jnp.float32), NPG, axis=0)
    SLOG = (jnp.concatenate([S, S], axis=1) - 1.0) * 100.0
    SBF = S.astype(jnp.bfloat16)
    ri, rv = _attn_readout(feat_invar, feat_var, gi, gv,
                           Wu, bu2, Wv, We, SLOG, SBF)
    return (ri.reshape(B, D)[:, None, :], rv.reshape(B, D)[:, None, :])
